# Initial kernel scaffold; baseline (speedup 1.0000x reference)
import jax
import jax.numpy as jnp
from jax.experimental import pallas as pl


def kernel(x, router_prob, q_w, q_b, k_w, k_b, v_w, v_b, o_w, o_b, ln_w, ln_b):
    raise NotImplementedError("write your pallas kernel here")


# speedup vs baseline: 1.1144x; 1.1144x over previous
import functools
import jax
import jax.numpy as jnp
from jax.experimental import pallas as pl

B = 2
T = 2048
D = 2048
E = 8
H = 8
N = T // E
DH = D // H
SCALE = D ** (-0.5)


def _greedy_route(probs):
    rp = probs
    idxs = []
    for e in range(E):
        _, idx = jax.lax.top_k(rp[:, :, e], N)
        idxs.append(idx)
        mask = jnp.zeros((B, T), dtype=bool).at[jnp.arange(B)[:, None], idx].set(True)
        rp = jnp.where(mask[:, :, None], 0.0, rp)
    return jnp.concatenate(idxs, axis=1)


def _qkv_body(x_ref, qw_ref, kw_ref, vw_ref, qb_ref, kb_ref, vb_ref,
              lnw_ref, lnb_ref, q_ref, k_ref, v_ref, *, m):
    xb = x_ref[0]
    mu = jnp.mean(xb, axis=1, keepdims=True)
    var = jnp.mean((xb - mu) ** 2, axis=1, keepdims=True)
    ln = (xb - mu) / jnp.sqrt(var + 1e-5) * lnw_ref[...] + lnb_ref[...]
    ex = ln[:, :m]
    dn = (((1,), (1,)), ((), ()))
    q_ref[0] = jax.lax.dot_general(ex, qw_ref[...], dn,
                                   preferred_element_type=jnp.float32) + qb_ref[...]
    k_ref[0] = jax.lax.dot_general(ex, kw_ref[...], dn,
                                   preferred_element_type=jnp.float32) + kb_ref[...]
    v_ref[0] = jax.lax.dot_general(ex, vw_ref[...], dn,
                                   preferred_element_type=jnp.float32) + vb_ref[...]


def _qkv_expert(xg_e, q_w, k_w, v_w, q_b, k_b, v_b, ln_w, ln_b, m):
    c = {2048: 4, 1024: 2}.get(m, 1)
    dout = D // c
    xspec = pl.BlockSpec((1, N, D), lambda b, j: (b, 0, 0))
    wspec = pl.BlockSpec((dout, m), lambda b, j: (j, 0))
    bspec = pl.BlockSpec((dout,), lambda b, j: (j,))
    lspec = pl.BlockSpec((D,), lambda b, j: (0,))
    ospec = pl.BlockSpec((1, N, dout), lambda b, j: (b, 0, j))
    out_sd = jax.ShapeDtypeStruct((B, N, D), jnp.float32)
    return pl.pallas_call(
        functools.partial(_qkv_body, m=m),
        grid=(B, c),
        in_specs=[xspec, wspec, wspec, wspec, bspec, bspec, bspec, lspec, lspec],
        out_specs=[ospec, ospec, ospec],
        out_shape=[out_sd, out_sd, out_sd],
    )(xg_e, q_w[:, :m], k_w[:, :m], v_w[:, :m], q_b, k_b, v_b, ln_w, ln_b)


def _attn_body(q_ref, k_ref, v_ref, o_ref):
    q = q_ref[0]
    k = k_ref[0]
    v = v_ref[0]
    s = jax.lax.dot_general(q, k, (((1,), (1,)), ((), ())),
                            preferred_element_type=jnp.float32) * SCALE
    mx = jnp.max(s, axis=1, keepdims=True)
    p = jnp.exp(s - mx)
    p = p / jnp.sum(p, axis=1, keepdims=True)
    o_ref[0] = jax.lax.dot_general(p, v, (((1,), (0,)), ((), ())),
                                   preferred_element_type=jnp.float32)


def _attention(q, k, v, bq=256):
    qspec = pl.BlockSpec((1, bq, DH), lambda b, h, i: (b, i, h))
    kvspec = pl.BlockSpec((1, T, DH), lambda b, h, i: (b, 0, h))
    return pl.pallas_call(
        _attn_body,
        grid=(B, H, T // bq),
        in_specs=[qspec, kvspec, kvspec],
        out_specs=qspec,
        out_shape=jax.ShapeDtypeStruct((B, T, D), jnp.float32),
    )(q, k, v)


def _oproj_body(a_ref, x_ref, ow_ref, ob_ref, o_ref, *, m):
    ab = a_ref[0]
    xb = x_ref[0]
    ex = ab[:, :m]
    proj = jax.lax.dot_general(ex, ow_ref[...], (((1,), (1,)), ((), ())),
                               preferred_element_type=jnp.float32) + ob_ref[...]
    if m == D:
        o_ref[0] = xb + proj
    else:
        o_ref[0] = jnp.concatenate([xb[:, :m] + proj, xb[:, m:]], axis=1)


def _oproj_expert(attn_e, x_e, o_w, o_b, m):
    full = pl.BlockSpec((1, N, D), lambda b: (b, 0, 0))
    wspec = pl.BlockSpec((m, m), lambda b: (0, 0))
    bspec = pl.BlockSpec((m,), lambda b: (0,))
    return pl.pallas_call(
        functools.partial(_oproj_body, m=m),
        grid=(B,),
        in_specs=[full, full, wspec, bspec],
        out_specs=full,
        out_shape=jax.ShapeDtypeStruct((B, N, D), jnp.float32),
    )(attn_e, x_e, o_w[:m, :m], o_b[:m])


def kernel(x, router_prob, q_w, q_b, k_w, k_b, v_w, v_b, o_w, o_b, ln_w, ln_b):
    perm = _greedy_route(router_prob)
    new_probs = jnp.take_along_axis(router_prob, perm[:, :, None], axis=1)
    xg = jnp.take_along_axis(x, perm[:, :, None], axis=1)

    qs, ks_, vs = [], [], []
    for e in range(E):
        m = D >> e
        qe, ke, ve = _qkv_expert(xg[:, e * N:(e + 1) * N], q_w, k_w, v_w,
                                 q_b, k_b, v_b, ln_w, ln_b, m)
        qs.append(qe); ks_.append(ke); vs.append(ve)
    q = jnp.concatenate(qs, axis=1)
    k = jnp.concatenate(ks_, axis=1)
    v = jnp.concatenate(vs, axis=1)

    attn_out = _attention(q, k, v)

    perm2 = _greedy_route(new_probs)
    attn_g = jnp.take_along_axis(attn_out, perm2[:, :, None], axis=1)
    x_g = jnp.take_along_axis(x, perm2[:, :, None], axis=1)
    final_probs = jnp.take_along_axis(new_probs, perm2[:, :, None], axis=1)

    outs = []
    for e in range(E):
        m = D >> e
        outs.append(_oproj_expert(attn_g[:, e * N:(e + 1) * N],
                                  x_g[:, e * N:(e + 1) * N], o_w, o_b, m))
    return jnp.concatenate(outs, axis=1), final_probs
